# Initial kernel scaffold; baseline (speedup 1.0000x reference)
import jax
import jax.numpy as jnp
from jax.experimental import pallas as pl


def kernel(x, edge_index, edge_attr, Wk, bk, Wq, bq, Wv, bv, Wskip, bias):
    raise NotImplementedError("write your pallas kernel here")


# speedup vs baseline: 1.1786x; 1.1786x over previous
import functools

import jax
import jax.numpy as jnp
from jax import lax
from jax.experimental import pallas as pl
from jax.experimental.pallas import tpu as pltpu
from jax.experimental.pallas import tpu_sc as plsc


def _node_proj_body(x_ref, wk_ref, wq_ref, wv_ref, ws_ref, bias_ref,
                    a_ref, bc_ref, s_ref):
    xb = x_ref[...]
    f32 = jnp.float32
    a_ref[...] = jnp.dot(xb, wk_ref[...], preferred_element_type=f32)
    bc_ref[:, : wq_ref.shape[1]] = jnp.dot(xb, wq_ref[...], preferred_element_type=f32)
    bc_ref[:, wq_ref.shape[1]:] = jnp.dot(xb, wv_ref[...], preferred_element_type=f32)
    s_ref[...] = jnp.dot(xb, ws_ref[...], preferred_element_type=f32) + bias_ref[...]


def _node_proj(x, wk, wq, wv, ws, bias):
    n, d = x.shape
    blk = 1000
    grid = n // blk
    full = lambda i: (0, 0)
    return pl.pallas_call(
        _node_proj_body,
        grid=(grid,),
        in_specs=[
            pl.BlockSpec((blk, d), lambda i: (i, 0)),
            pl.BlockSpec((d, d), full),
            pl.BlockSpec((d, d), full),
            pl.BlockSpec((d, d), full),
            pl.BlockSpec((d, d), full),
            pl.BlockSpec((1, d), full),
        ],
        out_specs=[
            pl.BlockSpec((blk, d), lambda i: (i, 0)),
            pl.BlockSpec((blk, 2 * d), lambda i: (i, 0)),
            pl.BlockSpec((blk, d), lambda i: (i, 0)),
        ],
        out_shape=[
            jax.ShapeDtypeStruct((n, d), jnp.float32),
            jax.ShapeDtypeStruct((n, 2 * d), jnp.float32),
            jax.ShapeDtypeStruct((n, d), jnp.float32),
        ],
    )(x, wk, wq, wv, ws, bias)


def _edge_proj_body(ea_ref, wke_ref, wqe_ref, wve_ref, bkq_ref, bv_ref, gh_ref):
    ea = ea_ref[...]
    f32 = jnp.float32
    d = wke_ref.shape[1]
    wg = wke_ref[...] + wqe_ref[...]
    gh_ref[:, :d] = jnp.dot(ea, wg, preferred_element_type=f32) + bkq_ref[...]
    gh_ref[:, d:] = jnp.dot(ea, wve_ref[...], preferred_element_type=f32) + bv_ref[...]


def _edge_proj(ea, wke, wqe, wve, bkq, bv):
    e, de = ea.shape
    d = wke.shape[1]
    blk = 4000
    grid = e // blk
    full = lambda i: (0, 0)
    return pl.pallas_call(
        _edge_proj_body,
        grid=(grid,),
        in_specs=[
            pl.BlockSpec((blk, de), lambda i: (i, 0)),
            pl.BlockSpec((de, d), full),
            pl.BlockSpec((de, d), full),
            pl.BlockSpec((de, d), full),
            pl.BlockSpec((1, d), full),
            pl.BlockSpec((1, d), full),
        ],
        out_specs=pl.BlockSpec((blk, 2 * d), lambda i: (i, 0)),
        out_shape=jax.ShapeDtypeStruct((e, 2 * d), jnp.float32),
    )(ea, wke, wqe, wve, bkq, bv)


def _add2_body(p0_ref, p1_ref, o_ref):
    o_ref[...] = p0_ref[...] + p1_ref[...]


def _add2(p0, p1):
    n, d = p0.shape
    blk = 1000
    return pl.pallas_call(
        _add2_body,
        grid=(n // blk,),
        in_specs=[pl.BlockSpec((blk, d), lambda i: (i, 0)),
                  pl.BlockSpec((blk, d), lambda i: (i, 0))],
        out_specs=pl.BlockSpec((blk, d), lambda i: (i, 0)),
        out_shape=jax.ShapeDtypeStruct((n, d), jnp.float32),
    )(p0, p1)


_NC, _NS = 2, 16
_BE = 40


def _sc_aggregate(n, d, e_total):
    nw = _NC * _NS
    per_w = e_total // nw
    nb = per_w // _BE
    rows_main = (n // _NS) // 8 * 8
    rows_last = n - (_NS - 1) * rows_main
    mesh = plsc.VectorSubcoreMesh(core_axis_name="c", subcore_axis_name="s")

    @functools.partial(
        pl.kernel,
        out_type=[jax.ShapeDtypeStruct((n, d), jnp.float32),
                  jax.ShapeDtypeStruct((n, d), jnp.float32)],
        mesh=mesh,
        scratch_types=[
            pltpu.VMEM((_BE,), jnp.int32),
            pltpu.VMEM((_BE,), jnp.int32),
            pltpu.VMEM((_BE, d), jnp.float32),
            pltpu.VMEM((_BE, 2 * d), jnp.float32),
            pltpu.VMEM((_BE, 2 * d), jnp.float32),
            pltpu.VMEM_SHARED((n, d), jnp.float32),
            pltpu.SemaphoreType.DMA,
            pltpu.SemaphoreType.DMA,
        ],
    )
    def agg(src_hbm, dst_hbm, a_hbm, bc_hbm, gh_hbm, s_hbm, z_hbm,
            out0_hbm, out1_hbm,
            srcv, dstv, av, bcv, ghv, acc, sem_a, sem_bc):
        cid = lax.axis_index("c")
        sid = lax.axis_index("s")
        wid = cid * _NS + sid
        row0 = sid * rows_main
        is_last = sid == _NS - 1

        def init_rows(nrows):
            @pl.when(cid == 0)
            def _():
                pltpu.sync_copy(s_hbm.at[pl.ds(row0, nrows)],
                                acc.at[pl.ds(row0, nrows)])

            @pl.when(cid != 0)
            def _():
                pltpu.sync_copy(z_hbm.at[pl.ds(row0, nrows)],
                                acc.at[pl.ds(row0, nrows)])

        @pl.when(jnp.logical_not(is_last))
        def _():
            init_rows(rows_main)

        @pl.when(is_last)
        def _():
            init_rows(rows_last)

        plsc.subcore_barrier()

        def batch(i, carry):
            base = wid * per_w + i * _BE
            pltpu.sync_copy(src_hbm.at[pl.ds(base, _BE)], srcv)
            pltpu.sync_copy(dst_hbm.at[pl.ds(base, _BE)], dstv)
            cp_a = pltpu.async_copy(a_hbm.at[dstv], av, sem_a)
            cp_bc = pltpu.async_copy(bc_hbm.at[srcv], bcv, sem_bc)
            pltpu.sync_copy(gh_hbm.at[pl.ds(base, _BE)], ghv)
            cp_a.wait()
            cp_bc.wait()

            def edge(eidx, c2):
                for j in range(d // 16):
                    o = j * 16
                    a = av[eidx, pl.ds(o, 16)]
                    b = bcv[eidx, pl.ds(o, 16)]
                    cvec = bcv[eidx, pl.ds(d + o, 16)]
                    g = ghv[eidx, pl.ds(o, 16)]
                    h = ghv[eidx, pl.ds(d + o, 16)]
                    gate = 1.0 / (1.0 + jnp.exp(-(a + b + g)))
                    av[eidx, pl.ds(o, 16)] = gate * (cvec + h)
                return c2

            lax.fori_loop(0, _BE, edge, 0)
            pltpu.sync_copy(av, acc.at[dstv], add=True)
            return carry

        lax.fori_loop(0, nb, batch, 0)
        plsc.subcore_barrier()

        def write_rows(nrows):
            @pl.when(cid == 0)
            def _():
                pltpu.sync_copy(acc.at[pl.ds(row0, nrows)],
                                out0_hbm.at[pl.ds(row0, nrows)])

            @pl.when(cid != 0)
            def _():
                pltpu.sync_copy(acc.at[pl.ds(row0, nrows)],
                                out1_hbm.at[pl.ds(row0, nrows)])

        @pl.when(jnp.logical_not(is_last))
        def _():
            write_rows(rows_main)

        @pl.when(is_last)
        def _():
            write_rows(rows_last)

    return agg


def kernel(x, edge_index, edge_attr, Wk, bk, Wq, bq, Wv, bv, Wskip, bias):
    n, d = x.shape
    e = edge_index.shape[1]

    a_tab, bc_tab, s_tab = _node_proj(
        x, Wk[:d], Wq[:d], Wv[:d], Wskip, bias.reshape(1, d))
    gh_tab = _edge_proj(
        edge_attr, Wk[d:], Wq[d:], Wv[d:],
        (bk + bq).reshape(1, d), bv.reshape(1, d))

    src = edge_index[0]
    dst = edge_index[1]
    zeros = jnp.zeros((n, d), jnp.float32)
    p0, p1 = _sc_aggregate(n, d, e)(src, dst, a_tab, bc_tab, gh_tab, s_tab, zeros)
    return _add2(p0, p1)
